# Initial kernel scaffold; baseline (speedup 1.0000x reference)
import jax
import jax.numpy as jnp
from jax.experimental import pallas as pl


def kernel(A, hypergraph_adj, adj, hypergraph_khop_and_k_shell, BC, params):
    raise NotImplementedError("write your pallas kernel here")


# speedup vs baseline: 5.4105x; 5.4105x over previous
import jax
import jax.numpy as jnp
from jax import lax
from jax.experimental import pallas as pl
from jax.experimental.pallas import tpu as pltpu

N = 1000
E = 1000
IN_F = 128
N_HID = 64
OUT_F = 32
END_F = 2
DEG_SIZE = 64
LINE_LEN = 999
KNN_K = 10
FC_DIM = N * 2 * END_F
FC_BLK = 512

_NEG = -1e9


def _leaky(x):
    return jnp.where(x >= 0, x, 0.2 * x)


def _c00(a, b):
    return lax.dot_general(a, b, (((0,), (0,)), ((), ())))


def _c11(a, b):
    return lax.dot_general(a, b, (((1,), (1,)), ((), ())))


def _gat_body(adj_ref, bct_ref, wbc1_ref, bbc1_ref, wbc2_ref, bbc2_ref,
              degt_ref, wg1_ref, a1s_ref, a1d_ref, wg2_ref, a2s_ref,
              a2d_ref, wl_ref, bl_ref, x_ref, gat_ref):
    adj = adj_ref[...]
    hbc = jnp.maximum(bct_ref[...] @ wbc1_ref[...] + bbc1_ref[...], 0.0)
    bc_f = hbc @ wbc2_ref[...] + bbc2_ref[...]
    deg = jnp.clip(jnp.sum(adj, axis=1, keepdims=True).astype(jnp.int32),
                   0, DEG_SIZE - 1)
    buckets = lax.broadcasted_iota(jnp.int32, (1, DEG_SIZE), 1)
    onehot = (deg == buckets).astype(jnp.float32)
    x_deg = onehot @ degt_ref[...]
    x = jnp.concatenate([x_deg, bc_f], axis=1)
    x_ref[...] = x

    mask = adj > 0

    def gat_layer(h, a_s_row, a_d_row):
        s_col = jnp.sum(h * a_s_row, axis=1, keepdims=True)
        t_row = _c11(a_d_row, h)
        e = _leaky(s_col + t_row)
        e = jnp.where(mask, e, _NEG)
        e = e - jnp.max(e, axis=1, keepdims=True)
        p = jnp.exp(e)
        num = p @ h
        den = jnp.sum(p, axis=1, keepdims=True)
        return num / den

    h1 = x @ wg1_ref[...]
    g1 = gat_layer(h1, a1s_ref[...], a1d_ref[...])
    g1 = jnp.where(g1 > 0, g1, jnp.exp(g1) - 1.0)
    h2 = g1 @ wg2_ref[...]
    g2 = gat_layer(h2, a2s_ref[...], a2d_ref[...])
    gat_ref[...] = g2 @ wl_ref[...] + bl_ref[...]


def _hyper_body(hga_ref, x_ref, wh1_ref, an1_ref, ae1_ref, wh2_ref,
                an2_ref, ae2_ref, wl2_ref, bl2_ref, emb_ref, emb0_ref):
    mask_t = hga_ref[...] > 0
    ones_col_e = jnp.ones((E, 1), jnp.float32)
    ones_col_n = jnp.ones((N, 1), jnp.float32)

    def layer(x1, an_row, ae_row, edge_contrib):
        s_n_row = _leaky(_c11(an_row, x1))
        le = jnp.where(mask_t, s_n_row, _NEG)
        le = le - jnp.max(le, axis=1, keepdims=True)
        pe = jnp.exp(le)
        ef = (pe @ x1) / (pe @ ones_col_n)
        if edge_contrib is not None:
            ef = ef + edge_contrib
        s_e_col = _leaky(jnp.sum(ef * ae_row, axis=1, keepdims=True))
        ln = jnp.where(mask_t, s_e_col, _NEG)
        ln = ln - jnp.max(ln, axis=0, keepdims=True)
        q = jnp.exp(ln)
        node = _c00(q, ef) / _c00(q, ones_col_e)
        return node, ef

    x1a = x_ref[...] @ wh1_ref[...]
    n1, ef1 = layer(x1a, an1_ref[...], ae1_ref[...], None)
    x1b = n1 @ wh2_ref[...]
    n2, _ = layer(x1b, an2_ref[...], ae2_ref[...], ef1 @ wh2_ref[...])
    emb_ref[...] = n2
    emb0_ref[...] = n2 @ wl2_ref[...] + bl2_ref[...]


def _knn_body(xemb_ref, w31_ref, a31n_ref, a31e_ref, w32_ref, a32n_ref,
              a32e_ref, hyp3_ref):
    xe = xemb_ref[...]
    sq = xe * xe
    d_col = jnp.sum(sq, axis=1, keepdims=True)
    d_row = _c11(jnp.ones((1, OUT_F), jnp.float32), sq)
    g = _c11(xe, xe)
    v = -(d_col + d_row - 2.0 * g)
    sent = -3e38
    jidx = lax.broadcasted_iota(jnp.int32, (N, N), 1)
    for _ in range(KNN_K):
        m = jnp.max(v, axis=1, keepdims=True)
        cand = jnp.where(v == m, jidx, N)
        jstar = jnp.min(cand, axis=1, keepdims=True)
        v = jnp.where(jidx == jstar, sent, v)
    mask = v == sent

    ones_col = jnp.ones((N, 1), jnp.float32)

    def layer(x1, an_row, ae_row, edge_contrib):
        s_n_col = _leaky(jnp.sum(x1 * an_row, axis=1, keepdims=True))
        le = jnp.where(mask, s_n_col, _NEG)
        le = le - jnp.max(le, axis=0, keepdims=True)
        pe = jnp.exp(le)
        ef = _c00(pe, x1) / _c00(pe, ones_col)
        if edge_contrib is not None:
            ef = ef + edge_contrib
        s_e_row = _leaky(_c11(ae_row, ef))
        ln = jnp.where(mask, s_e_row, _NEG)
        ln = ln - jnp.max(ln, axis=1, keepdims=True)
        pn = jnp.exp(ln)
        node = (pn @ ef) / (pn @ ones_col)
        return node, ef

    x3 = xe @ w31_ref[...]
    m1, f1 = layer(x3, a31n_ref[...], a31e_ref[...], None)
    x4 = m1 @ w32_ref[...]
    m2, _ = layer(x4, a32n_ref[...], a32e_ref[...], f1 @ w32_ref[...])
    hyp3_ref[...] = m2


def _head_body(emb_ref, wfc_ref, bfc_ref, wfc3_ref, bfc3_ref, out_ref):
    j = pl.program_id(0)

    @pl.when(j == 0)
    def _():
        out_ref[...] = jnp.zeros_like(out_ref)

    h = emb_ref[...] @ wfc_ref[...] + bfc_ref[...]
    h = jnp.where(h >= 0, h, 0.01 * h)
    col = j * FC_BLK + lax.broadcasted_iota(jnp.int32, (1, FC_BLK), 1)
    h = jnp.where(col < FC_DIM, h, 0.0)
    w3 = wfc3_ref[...]
    rowi = j * FC_BLK + lax.broadcasted_iota(jnp.int32, (FC_BLK, 1), 0)
    w3 = jnp.where(rowi < FC_DIM, w3, 0.0)
    out_ref[...] += h @ w3

    @pl.when(j == pl.num_programs(0) - 1)
    def _():
        out_ref[...] = jax.nn.sigmoid(out_ref[...] + bfc3_ref[...])


def _vmem_params():
    return pltpu.CompilerParams(vmem_limit_bytes=100 * 1024 * 1024)


def kernel(A, hypergraph_adj, adj, hypergraph_khop_and_k_shell, BC, params):
    p = params
    row = lambda v: v.reshape(1, -1)

    x, gat_out = pl.pallas_call(
        _gat_body,
        out_shape=[jax.ShapeDtypeStruct((N, IN_F), jnp.float32),
                   jax.ShapeDtypeStruct((N, 1), jnp.float32)],
        compiler_params=_vmem_params(),
    )(adj, BC.T, p["W_bc1"], row(p["b_bc1"]), p["W_bc2"], row(p["b_bc2"]),
      p["deg_table"], p["Wg1"], row(p["a1s"]), row(p["a1d"]), p["Wg2"],
      row(p["a2s"]), row(p["a2d"]), p["Wl"], row(p["bl"]))

    hyp_emb, emb0 = pl.pallas_call(
        _hyper_body,
        out_shape=[jax.ShapeDtypeStruct((N, OUT_F), jnp.float32),
                   jax.ShapeDtypeStruct((N, END_F), jnp.float32)],
        compiler_params=_vmem_params(),
    )(hypergraph_adj, x, p["Wh1"], row(p["an1"]), row(p["ae1"]), p["Wh2"],
      row(p["an2"]), row(p["ae2"]), p["Wl2"], row(p["bl2"]))

    hyp3 = pl.pallas_call(
        _knn_body,
        out_shape=jax.ShapeDtypeStruct((N, 1), jnp.float32),
        compiler_params=_vmem_params(),
    )(hyp_emb, p["W31"], row(p["a31n"]), row(p["a31e"]), p["W32"],
      row(p["a32n"]), row(p["a32e"]))

    emb = jnp.concatenate([gat_out, emb0, hyp3], axis=1).reshape(1, -1)

    nblk = (FC_DIM + FC_BLK - 1) // FC_BLK
    out = pl.pallas_call(
        _head_body,
        grid=(nblk,),
        in_specs=[
            pl.BlockSpec((1, FC_DIM), lambda j: (0, 0)),
            pl.BlockSpec((FC_DIM, FC_BLK), lambda j: (0, j)),
            pl.BlockSpec((1, FC_BLK), lambda j: (0, j)),
            pl.BlockSpec((FC_BLK, LINE_LEN), lambda j: (j, 0)),
            pl.BlockSpec((1, LINE_LEN), lambda j: (0, 0)),
        ],
        out_specs=pl.BlockSpec((1, LINE_LEN), lambda j: (0, 0)),
        out_shape=jax.ShapeDtypeStruct((1, LINE_LEN), jnp.float32),
        compiler_params=_vmem_params(),
    )(emb, p["Wfc"], row(p["bfc"]), p["Wfc3"], row(p["bfc3"]))

    return out
